# baseline (speedup 1.0000x reference)
import functools

import jax
import jax.numpy as jnp
from jax import lax
from jax.experimental import pallas as pl
from jax.experimental.pallas import tpu as pltpu
from jax.experimental.pallas import tpu_sc as plsc

SRC_VOCAB = 1000000
TRG_VOCAB = 1000000
DIM = 32
BATCH = 16384

_INFO = plsc.get_sparse_core_info()
_NC = _INFO.num_cores
_NS = _INFO.num_subcores
_NW = _NC * _NS
_BPW = BATCH // _NW
_CHUNK = 128
_NCHUNK = _BPW // _CHUNK
_IDX_ROWS = BATCH // _CHUNK


def _gather_body(semb, temb, xidx, yidx, xout, yout,
                 xi_v, yi_v, xr_v, yr_v, sem):
    wid = lax.axis_index("s") * _NC + lax.axis_index("c")
    base = wid * _BPW
    idx_row0 = wid * _NCHUNK
    pltpu.sync_copy(xidx.at[pl.ds(idx_row0, _NCHUNK)], xi_v)
    pltpu.sync_copy(yidx.at[pl.ds(idx_row0, _NCHUNK)], yi_v)
    copies = []
    for j in range(_NCHUNK):
        copies.append(pltpu.async_copy(
            semb.at[xi_v.at[j]], xr_v.at[pl.ds(j * _CHUNK, _CHUNK)], sem))
    for j in range(_NCHUNK):
        copies.append(pltpu.async_copy(
            temb.at[yi_v.at[j]], yr_v.at[pl.ds(j * _CHUNK, _CHUNK)], sem))
    for c in copies:
        c.wait()
    pltpu.sync_copy(xr_v, xout.at[pl.ds(base, _BPW)])
    pltpu.sync_copy(yr_v, yout.at[pl.ds(base, _BPW)])


@jax.jit
def _gather(semb, temb, xidx2d, yidx2d):
    mesh = plsc.VectorSubcoreMesh(core_axis_name="c", subcore_axis_name="s")
    f32 = jnp.float32
    return pl.kernel(
        _gather_body,
        mesh=mesh,
        out_type=[jax.ShapeDtypeStruct((BATCH, DIM), f32),
                  jax.ShapeDtypeStruct((BATCH, DIM), f32)],
        scratch_types=[
            pltpu.VMEM((_NCHUNK, _CHUNK), jnp.int32),
            pltpu.VMEM((_NCHUNK, _CHUNK), jnp.int32),
            pltpu.VMEM((_BPW, DIM), f32),
            pltpu.VMEM((_BPW, DIM), f32),
            pltpu.SemaphoreType.DMA,
        ],
        compiler_params=pltpu.CompilerParams(use_tc_tiling_on_sc=False),
    )(semb, temb, xidx2d, yidx2d)


def _proj_body(xe, ye, mw, mpw, xo, yo):
    dn = (((1,), (1,)), ((), ()))
    xo[...] = lax.dot_general(xe[...], mw[...], dn,
                              preferred_element_type=jnp.float32)
    yo[...] = lax.dot_general(ye[...], mpw[...], dn,
                              preferred_element_type=jnp.float32)


@jax.jit
def _project(xe, ye, m_W, mp_W):
    nblk = 8
    blk = BATCH // nblk
    f32 = jnp.float32
    row_spec = pl.BlockSpec((blk, DIM), lambda i: (i, 0))
    w_spec = pl.BlockSpec((DIM, DIM), lambda i: (0, 0))
    return pl.pallas_call(
        _proj_body,
        grid=(nblk,),
        in_specs=[row_spec, row_spec, w_spec, w_spec],
        out_specs=[row_spec, row_spec],
        out_shape=[jax.ShapeDtypeStruct((BATCH, DIM), f32),
                   jax.ShapeDtypeStruct((BATCH, DIM), f32)],
    )(xe, ye, m_W, mp_W)


def kernel(x_idx, y_idx, semb, temb, m_W, mp_W):
    xidx2d = x_idx.astype(jnp.int32).reshape(_IDX_ROWS, _CHUNK)
    yidx2d = y_idx.astype(jnp.int32).reshape(_IDX_ROWS, _CHUNK)
    xe, ye = _gather(semb, temb, xidx2d, yidx2d)
    xp, yp = _project(xe, ye, m_W, mp_W)
    return (xp, yp)


# speedup vs baseline: 1.5872x; 1.5872x over previous
import functools

import jax
import jax.numpy as jnp
from jax import lax
from jax.experimental import pallas as pl
from jax.experimental.pallas import tpu as pltpu
from jax.experimental.pallas import tpu_sc as plsc

DIM = 32
BATCH = 16384
VOCAB = 1000000
_PBLK = 16384
_SUB = _PBLK // 4
_PN = -(-VOCAB // _PBLK)
LINES = _PN * _SUB

_INFO = plsc.get_sparse_core_info()
_NC = _INFO.num_cores
_NS = _INFO.num_subcores
_NW = _NC * _NS
_BPW = BATCH // _NW
_CH = 128
_NCH = _BPW // _CH

def _proj_body(xT, yT, mw, mpw, ps, pt):
    dn = (((0,), (1,)), ((), ()))
    for src, w, out in ((xT, mw, ps), (yT, mpw, pt)):
        blk = src[...]
        out[...] = jnp.concatenate(
            [lax.dot_general(blk[:, q * _SUB:(q + 1) * _SUB], w[...], dn,
                             preferred_element_type=jnp.float32)
             for q in range(4)], axis=1)


@jax.jit
def _project(sembT, tembT, m_W, mp_W):
    f32 = jnp.float32
    in_spec = pl.BlockSpec((DIM, _PBLK), lambda i: (0, i))
    w_spec = pl.BlockSpec((DIM, DIM), lambda i: (0, 0))
    out_spec = pl.BlockSpec((_SUB, 128), lambda i: (i, 0))
    return pl.pallas_call(
        _proj_body,
        grid=(_PN,),
        in_specs=[in_spec, in_spec, w_spec, w_spec],
        out_specs=[out_spec, out_spec],
        out_shape=[jax.ShapeDtypeStruct((LINES, 128), f32),
                   jax.ShapeDtypeStruct((LINES, 128), f32)],
    )(sembT, tembT, m_W, mp_W)


def _gather_body(ps, pt, xidx, yidx, xl, yl, xi_v, yi_v, lx_v, ly_v,
                 gbuf, sem):
    wid = lax.axis_index("s") * _NC + lax.axis_index("c")
    base = wid * _BPW
    pltpu.sync_copy(xidx, xi_v)
    pltpu.sync_copy(yidx, yi_v)

    def mk_lines(m, _):
        rx = xi_v[pl.ds(base + m * 16, 16)]
        ry = yi_v[pl.ds(base + m * 16, 16)]
        lx_v[pl.ds(m * 16, 16)] = ((rx >> 14) << 12) + (rx & (_SUB - 1))
        ly_v[pl.ds(m * 16, 16)] = ((ry >> 14) << 12) + (ry & (_SUB - 1))
        return _

    lax.fori_loop(0, _BPW // 16, mk_lines, None)


    def launch(k):
        tab, idx, out = ((ps, lx_v, xl) if k < _NCH else (pt, ly_v, yl))
        j = k % _NCH
        return (tab.at[idx.at[pl.ds(j * _CH, _CH)]],
                gbuf.at[k % 2], out.at[pl.ds(base + j * _CH, _CH), :])

    for k in range(2 * _NCH):
        src, slot, dst = launch(k)
        if k >= 2:
            _, pslot, pdst = launch(k - 2)
            pltpu.make_async_copy(ps.at[pl.ds(0, _CH), :], pslot, sem).wait()
            pltpu.sync_copy(pslot, pdst)
        pltpu.async_copy(src, slot, sem)
    for k in (2 * _NCH - 2, 2 * _NCH - 1):
        _, slot, dst = launch(k)
        pltpu.make_async_copy(ps.at[pl.ds(0, _CH), :], slot, sem).wait()
        pltpu.sync_copy(slot, dst)


@jax.jit
def _gather(ps, pt, xidx, yidx):
    mesh = plsc.VectorSubcoreMesh(core_axis_name="c", subcore_axis_name="s")
    f32 = jnp.float32
    return pl.kernel(
        _gather_body,
        mesh=mesh,
        out_type=[jax.ShapeDtypeStruct((BATCH, 128), f32),
                  jax.ShapeDtypeStruct((BATCH, 128), f32)],
        scratch_types=[
            pltpu.VMEM((BATCH,), jnp.int32),
            pltpu.VMEM((BATCH,), jnp.int32),
            pltpu.VMEM((_BPW,), jnp.int32),
            pltpu.VMEM((_BPW,), jnp.int32),
            pltpu.VMEM((2, _CH, 128), f32),
            pltpu.SemaphoreType.DMA,
        ],
    )(ps, pt, xidx, yidx)


_SBLK = 1024
_SN = BATCH // _SBLK


def _select_body(xl, yl, xq, yq, xoT, yoT):
    for lines, qref, out in ((xl, xq, xoT), (yl, yq, yoT)):
        q = (qref[...] >> 12) & 3
        l3 = lines[...].reshape(_SBLK, 4, DIM)
        acc = jnp.zeros((_SBLK, DIM), jnp.float32)
        for qi in range(4):
            acc = acc + jnp.where(q == qi, l3[:, qi, :], 0.0)
        out[...] = acc.T


@jax.jit
def _select(xl, yl, xq2, yq2):
    f32 = jnp.float32
    l_spec = pl.BlockSpec((_SBLK, 128), lambda i: (i, 0))
    q_spec = pl.BlockSpec((_SBLK, 1), lambda i: (i, 0))
    out_spec = pl.BlockSpec((DIM, _SBLK), lambda i: (0, i))
    return pl.pallas_call(
        _select_body,
        grid=(_SN,),
        in_specs=[l_spec, l_spec, q_spec, q_spec],
        out_specs=[out_spec, out_spec],
        out_shape=[jax.ShapeDtypeStruct((DIM, BATCH), f32),
                   jax.ShapeDtypeStruct((DIM, BATCH), f32)],
    )(xl, yl, xq2, yq2)


def kernel(x_idx, y_idx, semb, temb, m_W, mp_W):
    xidx = x_idx.astype(jnp.int32)
    yidx = y_idx.astype(jnp.int32)
    ps, pt = _project(semb.T, temb.T, m_W, mp_W)
    xl, yl = _gather(ps, pt, xidx, yidx)
    xpT, ypT = _select(xl, yl, xidx.reshape(BATCH, 1),
                       yidx.reshape(BATCH, 1))
    return (xpT.T, ypT.T)


# speedup vs baseline: 3.3528x; 2.1123x over previous
import functools

import jax
import jax.numpy as jnp
from jax import lax
from jax.experimental import pallas as pl
from jax.experimental.pallas import tpu as pltpu
from jax.experimental.pallas import tpu_sc as plsc

DIM = 32
BATCH = 16384
VOCAB = 1000000
_PBLK = 16384
_SUB = _PBLK // 4
_PN = -(-VOCAB // _PBLK)
LINES = _PN * _SUB

_INFO = plsc.get_sparse_core_info()
_NC = _INFO.num_cores
_NS = _INFO.num_subcores
_NW = _NC * _NS
_BPW = BATCH // _NW
_CH = 128
_NCH = _BPW // _CH

def _proj_body(xT, yT, mw, mpw, ps, pt):
    dn = (((0,), (0,)), ((), ()))
    for src, w, out in ((xT, mw, ps), (yT, mpw, pt)):
        blk = src[...]
        cat = jnp.concatenate(
            [blk[:, q * _SUB:(q + 1) * _SUB] for q in range(4)], axis=0)
        out[...] = lax.dot_general(cat, w[...], dn,
                                   preferred_element_type=jnp.float32)


@jax.jit
def _project(sembT, tembT, m_W, mp_W):
    f32 = jnp.float32
    in_spec = pl.BlockSpec((DIM, _PBLK), lambda i: (0, i))
    w_spec = pl.BlockSpec((128, 128), lambda i: (0, 0))
    out_spec = pl.BlockSpec((_SUB, 128), lambda i: (i, 0))
    return pl.pallas_call(
        _proj_body,
        grid=(_PN,),
        in_specs=[in_spec, in_spec, w_spec, w_spec],
        out_specs=[out_spec, out_spec],
        out_shape=[jax.ShapeDtypeStruct((LINES, 128), f32),
                   jax.ShapeDtypeStruct((LINES, 128), f32)],
    )(sembT, tembT, m_W, mp_W)


def _gather_body(ps, pt, xidx, yidx, xl, yl, xi_v, yi_v, lx_v, ly_v,
                 gbuf, sem):
    wid = lax.axis_index("s") * _NC + lax.axis_index("c")
    base = wid * _BPW
    pltpu.sync_copy(xidx, xi_v)
    pltpu.sync_copy(yidx, yi_v)

    def mk_lines(m, _):
        rx = xi_v[pl.ds(base + m * 16, 16)]
        ry = yi_v[pl.ds(base + m * 16, 16)]
        lx_v[pl.ds(m * 16, 16)] = ((rx >> 14) << 12) + (rx & (_SUB - 1))
        ly_v[pl.ds(m * 16, 16)] = ((ry >> 14) << 12) + (ry & (_SUB - 1))
        return _

    lax.fori_loop(0, _BPW // 16, mk_lines, None)


    def launch(k):
        tab, idx, out = ((ps, lx_v, xl) if k < _NCH else (pt, ly_v, yl))
        j = k % _NCH
        return (tab.at[idx.at[pl.ds(j * _CH, _CH)]],
                gbuf.at[k % 2], out.at[pl.ds(base + j * _CH, _CH), :])

    for k in range(2 * _NCH):
        src, slot, dst = launch(k)
        if k >= 2:
            _, pslot, pdst = launch(k - 2)
            pltpu.make_async_copy(ps.at[pl.ds(0, _CH), :], pslot, sem).wait()
            pltpu.sync_copy(pslot, pdst)
        pltpu.async_copy(src, slot, sem)
    for k in (2 * _NCH - 2, 2 * _NCH - 1):
        _, slot, dst = launch(k)
        pltpu.make_async_copy(ps.at[pl.ds(0, _CH), :], slot, sem).wait()
        pltpu.sync_copy(slot, dst)


@jax.jit
def _gather(ps, pt, xidx, yidx):
    mesh = plsc.VectorSubcoreMesh(core_axis_name="c", subcore_axis_name="s")
    f32 = jnp.float32
    return pl.kernel(
        _gather_body,
        mesh=mesh,
        out_type=[jax.ShapeDtypeStruct((BATCH, 128), f32),
                  jax.ShapeDtypeStruct((BATCH, 128), f32)],
        scratch_types=[
            pltpu.VMEM((BATCH,), jnp.int32),
            pltpu.VMEM((BATCH,), jnp.int32),
            pltpu.VMEM((_BPW,), jnp.int32),
            pltpu.VMEM((_BPW,), jnp.int32),
            pltpu.VMEM((2, _CH, 128), f32),
            pltpu.SemaphoreType.DMA,
        ],
    )(ps, pt, xidx, yidx)


_SBLK = 1024
_SN = BATCH // _SBLK


def _select_body(xl, yl, xq, yq, xoT, yoT):
    for lines, qref, out in ((xl, xq, xoT), (yl, yq, yoT)):
        q = (qref[...] >> 12) & 3
        l3 = lines[...].reshape(_SBLK, 4, DIM)
        acc = jnp.zeros((_SBLK, DIM), jnp.float32)
        for qi in range(4):
            acc = acc + jnp.where(q == qi, l3[:, qi, :], 0.0)
        eye = jnp.eye(DIM, dtype=jnp.float32)
        out[...] = lax.dot_general(eye, acc, (((1,), (1,)), ((), ())),
                                   preferred_element_type=jnp.float32)


@jax.jit
def _select(xl, yl, xq2, yq2):
    f32 = jnp.float32
    l_spec = pl.BlockSpec((_SBLK, 128), lambda i: (i, 0))
    q_spec = pl.BlockSpec((_SBLK, 1), lambda i: (i, 0))
    out_spec = pl.BlockSpec((DIM, _SBLK), lambda i: (0, i))
    return pl.pallas_call(
        _select_body,
        grid=(_SN,),
        in_specs=[l_spec, l_spec, q_spec, q_spec],
        out_specs=[out_spec, out_spec],
        out_shape=[jax.ShapeDtypeStruct((DIM, BATCH), f32),
                   jax.ShapeDtypeStruct((DIM, BATCH), f32)],
    )(xl, yl, xq2, yq2)


def kernel(x_idx, y_idx, semb, temb, m_W, mp_W):
    xidx = x_idx.astype(jnp.int32)
    yidx = y_idx.astype(jnp.int32)
    eye4 = jnp.eye(4, dtype=jnp.float32)
    ps, pt = _project(semb.T, temb.T,
                      jnp.kron(eye4, m_W.T), jnp.kron(eye4, mp_W.T))
    xl, yl = _gather(ps, pt, xidx, yidx)
    xpT, ypT = _select(xl, yl, xidx.reshape(BATCH, 1),
                       yidx.reshape(BATCH, 1))
    return (xpT.T, ypT.T)


# speedup vs baseline: 3.3874x; 1.0103x over previous
import functools

import jax
import jax.numpy as jnp
from jax import lax
from jax.experimental import pallas as pl
from jax.experimental.pallas import tpu as pltpu
from jax.experimental.pallas import tpu_sc as plsc

DIM = 32
BATCH = 16384
VOCAB = 1000000
_PBLK = 16384
_SUB = _PBLK // 4
_PN = -(-VOCAB // _PBLK)
LINES = _PN * _SUB

_INFO = plsc.get_sparse_core_info()
_NC = _INFO.num_cores
_NS = _INFO.num_subcores
_NW = _NC * _NS
_BPW = BATCH // _NW
_CH = 128
_NCH = _BPW // _CH

def _proj_body(xT, w, ps):
    dn = (((0,), (0,)), ((), ()))
    blk = xT[...]
    cat = jnp.concatenate(
        [blk[:, q * _SUB:(q + 1) * _SUB] for q in range(4)], axis=0)
    ps[...] = lax.dot_general(cat, w[...], dn,
                              preferred_element_type=jnp.float32)


@jax.jit
def _project(sembT, wbig):
    f32 = jnp.float32
    in_spec = pl.BlockSpec((DIM, _PBLK), lambda i: (0, i))
    w_spec = pl.BlockSpec((128, 128), lambda i: (0, 0))
    out_spec = pl.BlockSpec((_SUB, 128), lambda i: (i, 0))
    return pl.pallas_call(
        _proj_body,
        grid=(_PN,),
        in_specs=[in_spec, w_spec],
        out_specs=out_spec,
        out_shape=jax.ShapeDtypeStruct((LINES, 128), f32),
    )(sembT, wbig)


def _gather_body(ps, xidx, xl, xi_v, lx_v, gbuf, sem):
    wid = lax.axis_index("s") * _NC + lax.axis_index("c")
    base = wid * _BPW
    pltpu.sync_copy(xidx, xi_v)

    def mk_lines(m, _):
        rx = xi_v[pl.ds(base + m * 16, 16)]
        lx_v[pl.ds(m * 16, 16)] = ((rx >> 14) << 12) + (rx & (_SUB - 1))
        return _

    lax.fori_loop(0, _BPW // 16, mk_lines, None)

    def launch(k):
        return (ps.at[lx_v.at[pl.ds(k * _CH, _CH)]],
                gbuf.at[k % 2], xl.at[pl.ds(base + k * _CH, _CH), :])

    for k in range(_NCH):
        src, slot, dst = launch(k)
        if k >= 2:
            _, pslot, pdst = launch(k - 2)
            pltpu.make_async_copy(ps.at[pl.ds(0, _CH), :], pslot, sem).wait()
            pltpu.sync_copy(pslot, pdst)
        pltpu.async_copy(src, slot, sem)
    for k in (_NCH - 2, _NCH - 1):
        _, slot, dst = launch(k)
        pltpu.make_async_copy(ps.at[pl.ds(0, _CH), :], slot, sem).wait()
        pltpu.sync_copy(slot, dst)


@jax.jit
def _gather(ps, xidx):
    mesh = plsc.VectorSubcoreMesh(core_axis_name="c", subcore_axis_name="s")
    f32 = jnp.float32
    return pl.kernel(
        _gather_body,
        mesh=mesh,
        out_type=jax.ShapeDtypeStruct((BATCH, 128), f32),
        scratch_types=[
            pltpu.VMEM((BATCH,), jnp.int32),
            pltpu.VMEM((_BPW,), jnp.int32),
            pltpu.VMEM((2, _CH, 128), f32),
            pltpu.SemaphoreType.DMA,
        ],
    )(ps, xidx)


_SBLK = 1024
_SN = BATCH // _SBLK


def _select_body(xl, xq, xoT):
    q = (xq[...] >> 12) & 3
    l128 = xl[...]
    i128 = jnp.eye(128, dtype=jnp.float32)
    acc = jnp.zeros((DIM, _SBLK), jnp.float32)
    for qi in range(4):
        masked = l128 * jnp.where(q == qi, 1.0, 0.0)
        acc = acc + lax.dot_general(
            i128[:, qi * DIM:(qi + 1) * DIM], masked,
            (((0,), (1,)), ((), ())), preferred_element_type=jnp.float32)
    xoT[...] = acc


@jax.jit
def _select(xl, xq2):
    f32 = jnp.float32
    l_spec = pl.BlockSpec((_SBLK, 128), lambda i: (i, 0))
    q_spec = pl.BlockSpec((_SBLK, 1), lambda i: (i, 0))
    out_spec = pl.BlockSpec((DIM, _SBLK), lambda i: (0, i))
    return pl.pallas_call(
        _select_body,
        grid=(_SN,),
        in_specs=[l_spec, q_spec],
        out_specs=out_spec,
        out_shape=jax.ShapeDtypeStruct((DIM, BATCH), f32),
    )(xl, xq2)


def kernel(x_idx, y_idx, semb, temb, m_W, mp_W):
    xidx = x_idx.astype(jnp.int32)
    yidx = y_idx.astype(jnp.int32)
    eye4 = jnp.eye(4, dtype=jnp.float32)
    ps = _project(semb.T, jnp.kron(eye4, m_W.T))
    xl = _gather(ps, xidx)
    pt = _project(temb.T, jnp.kron(eye4, mp_W.T))
    yl = _gather(pt, yidx)
    xpT = _select(xl, xidx.reshape(BATCH, 1))
    ypT = _select(yl, yidx.reshape(BATCH, 1))
    return (xpT.T, ypT.T)


# speedup vs baseline: 3.7260x; 1.1000x over previous
import functools

import jax
import jax.numpy as jnp
from jax import lax
from jax.experimental import pallas as pl
from jax.experimental.pallas import tpu as pltpu
from jax.experimental.pallas import tpu_sc as plsc

DIM = 32
BATCH = 16384
VOCAB = 1000000
_PBLK = 16384
_SUB = _PBLK // 4
_PN = -(-VOCAB // _PBLK)
LINES = _PN * _SUB

_INFO = plsc.get_sparse_core_info()
_NC = _INFO.num_cores
_NS = _INFO.num_subcores
_NW = _NC * _NS
_BPW = BATCH // _NW
_CH = 128
_NCH = _BPW // _CH

def _proj_body(xT, w, ps):
    dn = (((0,), (0,)), ((), ()))
    blk = xT[...]
    cat = jnp.concatenate(
        [blk[:, q * _SUB:(q + 1) * _SUB] for q in range(4)], axis=0)
    ps[...] = lax.dot_general(cat, w[...], dn,
                              preferred_element_type=jnp.float32)


@jax.jit
def _project(sembT, wbig):
    f32 = jnp.float32
    in_spec = pl.BlockSpec((DIM, _PBLK), lambda i: (0, i))
    w_spec = pl.BlockSpec((128, 128), lambda i: (0, 0))
    out_spec = pl.BlockSpec((_SUB, 128), lambda i: (i, 0))
    return pl.pallas_call(
        _proj_body,
        grid=(_PN,),
        in_specs=[in_spec, w_spec],
        out_specs=out_spec,
        out_shape=jax.ShapeDtypeStruct((LINES, 128), f32),
    )(sembT, wbig)


def _gather_body(ps, xidx, xl, xi_v, lx_v, gbuf, sem):
    wid = lax.axis_index("s") * _NC + lax.axis_index("c")
    base = wid * _BPW
    pltpu.sync_copy(xidx, xi_v)

    def mk_lines(m, _):
        rx = xi_v[pl.ds(base + m * 16, 16)]
        lx_v[pl.ds(m * 16, 16)] = ((rx >> 14) << 12) + (rx & (_SUB - 1))
        return _

    lax.fori_loop(0, _BPW // 16, mk_lines, None)

    def launch(k):
        return (ps.at[lx_v.at[pl.ds(k * _CH, _CH)]],
                gbuf.at[k % 2], xl.at[pl.ds(base + k * _CH, _CH), :])

    for k in range(_NCH):
        src, slot, dst = launch(k)
        if k >= 2:
            _, pslot, pdst = launch(k - 2)
            pltpu.make_async_copy(ps.at[pl.ds(0, _CH), :], pslot, sem).wait()
            pltpu.sync_copy(pslot, pdst)
        pltpu.async_copy(src, slot, sem)
    for k in (_NCH - 2, _NCH - 1):
        _, slot, dst = launch(k)
        pltpu.make_async_copy(ps.at[pl.ds(0, _CH), :], slot, sem).wait()
        pltpu.sync_copy(slot, dst)


@jax.jit
def _gather(ps, xidx):
    mesh = plsc.VectorSubcoreMesh(core_axis_name="c", subcore_axis_name="s")
    f32 = jnp.float32
    return pl.kernel(
        _gather_body,
        mesh=mesh,
        out_type=jax.ShapeDtypeStruct((BATCH, 128), f32),
        scratch_types=[
            pltpu.VMEM((BATCH,), jnp.int32),
            pltpu.VMEM((_BPW,), jnp.int32),
            pltpu.VMEM((2, _CH, 128), f32),
            pltpu.SemaphoreType.DMA,
        ],
    )(ps, xidx)


_SBLK = 4096
_SN = BATCH // _SBLK


def _select_body(xl, xm, xoT):
    l128 = xl[...]
    m4 = xm[...]
    i128 = jnp.eye(128, dtype=jnp.float32)
    acc = jnp.zeros((DIM, _SBLK), jnp.float32)
    for qi in range(4):
        t = lax.dot_general(
            i128[:, qi * DIM:(qi + 1) * DIM], l128,
            (((0,), (1,)), ((), ())), preferred_element_type=jnp.float32)
        acc = acc + t * m4[qi:qi + 1, :]
    xoT[...] = acc


@jax.jit
def _select(xl, xm):
    f32 = jnp.float32
    l_spec = pl.BlockSpec((_SBLK, 128), lambda i: (i, 0))
    m_spec = pl.BlockSpec((4, _SBLK), lambda i: (0, i))
    out_spec = pl.BlockSpec((DIM, _SBLK), lambda i: (0, i))
    return pl.pallas_call(
        _select_body,
        grid=(_SN,),
        in_specs=[l_spec, m_spec],
        out_specs=out_spec,
        out_shape=jax.ShapeDtypeStruct((DIM, BATCH), f32),
    )(xl, xm)


def kernel(x_idx, y_idx, semb, temb, m_W, mp_W):
    xidx = x_idx.astype(jnp.int32)
    yidx = y_idx.astype(jnp.int32)
    eye4 = jnp.eye(4, dtype=jnp.float32)
    ps = _project(semb.T, jnp.kron(eye4, m_W.T))
    xl = _gather(ps, xidx)
    pt = _project(temb.T, jnp.kron(eye4, mp_W.T))
    yl = _gather(pt, yidx)
    qs = jnp.arange(4, dtype=jnp.int32)[:, None]
    xm = (((xidx >> 12) & 3)[None, :] == qs).astype(jnp.float32)
    ym = (((yidx >> 12) & 3)[None, :] == qs).astype(jnp.float32)
    xpT = _select(xl, xm)
    ypT = _select(yl, ym)
    return (xpT.T, ypT.T)


# speedup vs baseline: 4.2302x; 1.1353x over previous
import functools

import jax
import jax.numpy as jnp
from jax import lax
from jax.experimental import pallas as pl
from jax.experimental.pallas import tpu as pltpu
from jax.experimental.pallas import tpu_sc as plsc

DIM = 32
BATCH = 16384
VOCAB = 1000000
_PBLK = 32768
_SUB = _PBLK // 4
_PN = -(-VOCAB // _PBLK)
LINES = _PN * _SUB

_INFO = plsc.get_sparse_core_info()
_NC = _INFO.num_cores
_NS = _INFO.num_subcores
_NW = _NC * _NS
_BPW = BATCH // _NW
_CH = 128
_NCH = _BPW // _CH

def _proj_body(xT, w, ps):
    dn = (((0,), (0,)), ((), ()))
    blk = xT[...]
    cat = jnp.concatenate(
        [blk[:, q * _SUB:(q + 1) * _SUB] for q in range(4)], axis=0)
    ps[...] = lax.dot_general(cat, w[...], dn,
                              preferred_element_type=jnp.float32)


@jax.jit
def _project(sembT, wbig):
    f32 = jnp.float32
    in_spec = pl.BlockSpec((DIM, _PBLK), lambda i: (0, i))
    w_spec = pl.BlockSpec((128, 128), lambda i: (0, 0))
    out_spec = pl.BlockSpec((_SUB, 128), lambda i: (i, 0))
    return pl.pallas_call(
        _proj_body,
        grid=(_PN,),
        in_specs=[in_spec, w_spec],
        out_specs=out_spec,
        out_shape=jax.ShapeDtypeStruct((LINES, 128), f32),
    )(sembT, wbig)


def _gather_body(ps, xidx, xl, xi_v, lx_v, gbuf, sem):
    wid = lax.axis_index("s") * _NC + lax.axis_index("c")
    base = wid * _BPW
    pltpu.sync_copy(xidx, xi_v)

    def mk_lines(m, _):
        rx = xi_v[pl.ds(base + m * 16, 16)]
        lx_v[pl.ds(m * 16, 16)] = ((rx >> 15) << 13) + (rx & (_SUB - 1))
        return _

    lax.fori_loop(0, _BPW // 16, mk_lines, None)

    def launch(k):
        return (ps.at[lx_v.at[pl.ds(k * _CH, _CH)]],
                gbuf.at[k % 2], xl.at[pl.ds(base + k * _CH, _CH), :])

    for k in range(_NCH):
        src, slot, dst = launch(k)
        if k >= 2:
            _, pslot, pdst = launch(k - 2)
            pltpu.make_async_copy(ps.at[pl.ds(0, _CH), :], pslot, sem).wait()
            pltpu.sync_copy(pslot, pdst)
        pltpu.async_copy(src, slot, sem)
    for k in (_NCH - 2, _NCH - 1):
        _, slot, dst = launch(k)
        pltpu.make_async_copy(ps.at[pl.ds(0, _CH), :], slot, sem).wait()
        pltpu.sync_copy(slot, dst)


@jax.jit
def _gather(ps, xidx):
    mesh = plsc.VectorSubcoreMesh(core_axis_name="c", subcore_axis_name="s")
    f32 = jnp.float32
    return pl.kernel(
        _gather_body,
        mesh=mesh,
        out_type=jax.ShapeDtypeStruct((BATCH, 128), f32),
        scratch_types=[
            pltpu.VMEM((BATCH,), jnp.int32),
            pltpu.VMEM((_BPW,), jnp.int32),
            pltpu.VMEM((2, _CH, 128), f32),
            pltpu.SemaphoreType.DMA,
        ],
    )(ps, xidx)


_SBLK = 4096
_SN = BATCH // _SBLK


def _select_body(xl, xm, xoT):
    l128 = xl[...]
    m4 = xm[...]
    i128 = jnp.eye(128, dtype=jnp.float32)
    acc = jnp.zeros((DIM, _SBLK), jnp.float32)
    for qi in range(4):
        t = lax.dot_general(
            i128[:, qi * DIM:(qi + 1) * DIM], l128,
            (((0,), (1,)), ((), ())), preferred_element_type=jnp.float32)
        acc = acc + t * m4[qi:qi + 1, :]
    xoT[...] = acc


@jax.jit
def _select(xl, xm):
    f32 = jnp.float32
    l_spec = pl.BlockSpec((_SBLK, 128), lambda i: (i, 0))
    m_spec = pl.BlockSpec((4, _SBLK), lambda i: (0, i))
    out_spec = pl.BlockSpec((DIM, _SBLK), lambda i: (0, i))
    return pl.pallas_call(
        _select_body,
        grid=(_SN,),
        in_specs=[l_spec, m_spec],
        out_specs=out_spec,
        out_shape=jax.ShapeDtypeStruct((DIM, BATCH), f32),
    )(xl, xm)


def kernel(x_idx, y_idx, semb, temb, m_W, mp_W):
    xidx = x_idx.astype(jnp.int32)
    yidx = y_idx.astype(jnp.int32)
    eye4 = jnp.eye(4, dtype=jnp.float32)
    ps = _project(semb.T, jnp.kron(eye4, m_W.T))
    xl = _gather(ps, xidx)
    pt = _project(temb.T, jnp.kron(eye4, mp_W.T))
    yl = _gather(pt, yidx)
    qs = jnp.arange(4, dtype=jnp.int32)[:, None]
    xm = (((xidx >> 13) & 3)[None, :] == qs).astype(jnp.float32)
    ym = (((yidx >> 13) & 3)[None, :] == qs).astype(jnp.float32)
    xpT = _select(xl, xm)
    ypT = _select(yl, ym)
    return (xpT.T, ypT.T)


# speedup vs baseline: 4.2587x; 1.0067x over previous
import functools

import jax
import jax.numpy as jnp
from jax import lax
from jax.experimental import pallas as pl
from jax.experimental.pallas import tpu as pltpu
from jax.experimental.pallas import tpu_sc as plsc

DIM = 32
BATCH = 16384
VOCAB = 1000000
_PBLK = 65536
_SUB = _PBLK // 4
_PN = -(-VOCAB // _PBLK)
LINES = _PN * _SUB

_INFO = plsc.get_sparse_core_info()
_NC = _INFO.num_cores
_NS = _INFO.num_subcores
_NW = _NC * _NS
_BPW = BATCH // _NW
_CH = 128
_NCH = _BPW // _CH

def _proj_body(xT, w, ps):
    dn = (((0,), (0,)), ((), ()))
    blk = xT[...]
    cat = jnp.concatenate(
        [blk[:, q * _SUB:(q + 1) * _SUB] for q in range(4)], axis=0)
    ps[...] = lax.dot_general(cat, w[...], dn,
                              preferred_element_type=jnp.float32)


@jax.jit
def _project(sembT, wbig):
    f32 = jnp.float32
    in_spec = pl.BlockSpec((DIM, _PBLK), lambda i: (0, i))
    w_spec = pl.BlockSpec((128, 128), lambda i: (0, 0))
    out_spec = pl.BlockSpec((_SUB, 128), lambda i: (i, 0))
    return pl.pallas_call(
        _proj_body,
        grid=(_PN,),
        in_specs=[in_spec, w_spec],
        out_specs=out_spec,
        out_shape=jax.ShapeDtypeStruct((LINES, 128), f32),
    )(sembT, wbig)


def _gather_body(ps, xidx, xl, xi_v, lx_v, gbuf, sem):
    wid = lax.axis_index("s") * _NC + lax.axis_index("c")
    base = wid * _BPW
    pltpu.sync_copy(xidx, xi_v)

    def mk_lines(m, _):
        rx = xi_v[pl.ds(base + m * 16, 16)]
        lx_v[pl.ds(m * 16, 16)] = ((rx >> 16) << 14) + (rx & (_SUB - 1))
        return _

    lax.fori_loop(0, _BPW // 16, mk_lines, None)

    def launch(k):
        return (ps.at[lx_v.at[pl.ds(k * _CH, _CH)]],
                gbuf.at[k % 2], xl.at[pl.ds(base + k * _CH, _CH), :])

    for k in range(_NCH):
        src, slot, dst = launch(k)
        if k >= 2:
            _, pslot, pdst = launch(k - 2)
            pltpu.make_async_copy(ps.at[pl.ds(0, _CH), :], pslot, sem).wait()
            pltpu.sync_copy(pslot, pdst)
        pltpu.async_copy(src, slot, sem)
    for k in (_NCH - 2, _NCH - 1):
        _, slot, dst = launch(k)
        pltpu.make_async_copy(ps.at[pl.ds(0, _CH), :], slot, sem).wait()
        pltpu.sync_copy(slot, dst)


@jax.jit
def _gather(ps, xidx):
    mesh = plsc.VectorSubcoreMesh(core_axis_name="c", subcore_axis_name="s")
    f32 = jnp.float32
    return pl.kernel(
        _gather_body,
        mesh=mesh,
        out_type=jax.ShapeDtypeStruct((BATCH, 128), f32),
        scratch_types=[
            pltpu.VMEM((BATCH,), jnp.int32),
            pltpu.VMEM((_BPW,), jnp.int32),
            pltpu.VMEM((2, _CH, 128), f32),
            pltpu.SemaphoreType.DMA,
        ],
    )(ps, xidx)


_SBLK = 4096
_SN = BATCH // _SBLK


def _select_body(xl, xm, xoT):
    l128 = xl[...]
    m4 = xm[...]
    i128 = jnp.eye(128, dtype=jnp.float32)
    acc = jnp.zeros((DIM, _SBLK), jnp.float32)
    for qi in range(4):
        t = lax.dot_general(
            i128[:, qi * DIM:(qi + 1) * DIM], l128,
            (((0,), (1,)), ((), ())), preferred_element_type=jnp.float32)
        acc = acc + t * m4[qi:qi + 1, :]
    xoT[...] = acc


@jax.jit
def _select(xl, xm):
    f32 = jnp.float32
    l_spec = pl.BlockSpec((_SBLK, 128), lambda i: (i, 0))
    m_spec = pl.BlockSpec((4, _SBLK), lambda i: (0, i))
    out_spec = pl.BlockSpec((DIM, _SBLK), lambda i: (0, i))
    return pl.pallas_call(
        _select_body,
        grid=(_SN,),
        in_specs=[l_spec, m_spec],
        out_specs=out_spec,
        out_shape=jax.ShapeDtypeStruct((DIM, BATCH), f32),
    )(xl, xm)


def kernel(x_idx, y_idx, semb, temb, m_W, mp_W):
    xidx = x_idx.astype(jnp.int32)
    yidx = y_idx.astype(jnp.int32)
    eye4 = jnp.eye(4, dtype=jnp.float32)
    ps = _project(semb.T, jnp.kron(eye4, m_W.T))
    xl = _gather(ps, xidx)
    pt = _project(temb.T, jnp.kron(eye4, mp_W.T))
    yl = _gather(pt, yidx)
    qs = jnp.arange(4, dtype=jnp.int32)[:, None]
    xm = (((xidx >> 14) & 3)[None, :] == qs).astype(jnp.float32)
    ym = (((yidx >> 14) & 3)[None, :] == qs).astype(jnp.float32)
    xpT = _select(xl, xm)
    ypT = _select(yl, ym)
    return (xpT.T, ypT.T)
